# baseline (speedup 1.0000x reference)
import functools

import jax
import jax.numpy as jnp
from jax import lax
from jax.experimental import pallas as pl
from jax.experimental.pallas import tpu as pltpu
from jax.experimental.pallas import tpu_sc as plsc

D_MODEL = 1024
FFN = 2048
N_EXPERTS = 8
TOP_K = 2
TILE = 128
ROW_TILE = 256

SC_CORES = 2
SC_SUBCORES = 16
NW = SC_CORES * SC_SUBCORES


def _router_kernel(x_ref, gw_ref, b_ref, meta_ref, wa_ref, wb_ref, cnt_ref,
                   run_ref):
    t = pl.program_id(0)

    @pl.when(t == 0)
    def _():
        run_ref[...] = jnp.zeros_like(run_ref)

    x = x_ref[...]
    gw = gw_ref[...]
    logits = lax.dot_general(x, gw, (((1,), (1,)), ((), ())),
                             preferred_element_type=jnp.float32)
    m = jnp.max(logits, axis=-1, keepdims=True)
    e = jnp.exp(logits - m)
    b = logits + b_ref[...]
    lane = lax.broadcasted_iota(jnp.int32, b.shape, 1)
    v1 = jnp.max(b, axis=-1, keepdims=True)
    i1 = jnp.min(jnp.where(b == v1, lane, N_EXPERTS), axis=-1, keepdims=True)
    oh1 = (lane == i1).astype(jnp.float32)
    b2 = jnp.where(oh1 > 0, -1e30, b)
    v2 = jnp.max(b2, axis=-1, keepdims=True)
    i2 = jnp.min(jnp.where(b2 == v2, lane, N_EXPERTS), axis=-1, keepdims=True)
    oh2 = (lane == i2).astype(jnp.float32)
    e1 = jnp.sum(e * oh1, axis=-1, keepdims=True)
    e2 = jnp.sum(e * oh2, axis=-1, keepdims=True)
    s = e1 + e2
    w1 = e1 / s
    w2 = e2 / s
    maskf = oh1 + oh2
    r_i = lax.broadcasted_iota(jnp.int32, (ROW_TILE, ROW_TILE), 0)
    c_i = lax.broadcasted_iota(jnp.int32, (ROW_TILE, ROW_TILE), 1)
    tri = (r_i >= c_i).astype(jnp.float32)
    local = lax.dot_general(tri, maskf, (((1,), (0,)), ((), ())),
                            preferred_element_type=jnp.float32)
    run0 = run_ref[...][0:1, :]
    rank = local + run0 - 1.0
    new_run = run0 + local[ROW_TILE - 1:ROW_TILE, :]
    run_ref[...] = jnp.broadcast_to(new_run, (N_EXPERTS, N_EXPERTS))
    cnt_ref[...] = jnp.broadcast_to(new_run, (N_EXPERTS, N_EXPERTS))
    i1f = i1.astype(jnp.float32)
    i2f = i2.astype(jnp.float32)
    i_lo = jnp.minimum(i1f, i2f)
    i_hi = jnp.maximum(i1f, i2f)
    lo_is_1 = (i1 < i2).astype(jnp.float32)
    w_lo = lo_is_1 * w1 + (1.0 - lo_is_1) * w2
    w_hi = lo_is_1 * w2 + (1.0 - lo_is_1) * w1
    oh_lo = (lane.astype(jnp.float32) == i_lo).astype(jnp.float32)
    oh_hi = (lane.astype(jnp.float32) == i_hi).astype(jnp.float32)
    rank_lo = jnp.sum(rank * oh_lo, axis=-1, keepdims=True)
    rank_hi = jnp.sum(rank * oh_hi, axis=-1, keepdims=True)
    l0 = (lane == 0).astype(jnp.float32)
    l1 = (lane == 1).astype(jnp.float32)
    l2 = (lane == 2).astype(jnp.float32)
    l3 = (lane == 3).astype(jnp.float32)
    meta_ref[...] = l0 * rank_lo + l1 * rank_hi + l2 * i_lo + l3 * i_hi
    wa_ref[...] = jnp.broadcast_to(w_lo, (ROW_TILE, 16))
    wb_ref[...] = jnp.broadcast_to(w_hi, (ROW_TILE, 16))


def _run_router(flat_x, gate_w, expert_bias):
    n = flat_x.shape[0]
    grid = (n // ROW_TILE,)
    return pl.pallas_call(
        _router_kernel,
        grid=grid,
        in_specs=[
            pl.BlockSpec((ROW_TILE, D_MODEL), lambda t: (t, 0)),
            pl.BlockSpec((N_EXPERTS, D_MODEL), lambda t: (0, 0)),
            pl.BlockSpec((1, N_EXPERTS), lambda t: (0, 0)),
        ],
        out_specs=[
            pl.BlockSpec((ROW_TILE, N_EXPERTS), lambda t: (t, 0)),
            pl.BlockSpec((ROW_TILE, 16), lambda t: (t, 0)),
            pl.BlockSpec((ROW_TILE, 16), lambda t: (t, 0)),
            pl.BlockSpec((N_EXPERTS, N_EXPERTS), lambda t: (0, 0)),
        ],
        out_shape=[
            jax.ShapeDtypeStruct((n, N_EXPERTS), jnp.float32),
            jax.ShapeDtypeStruct((n, 16), jnp.float32),
            jax.ShapeDtypeStruct((n, 16), jnp.float32),
            jax.ShapeDtypeStruct((N_EXPERTS, N_EXPERTS), jnp.float32),
        ],
        scratch_shapes=[pltpu.VMEM((N_EXPERTS, N_EXPERTS), jnp.float32)],
    )(flat_x, gate_w, expert_bias.reshape(1, N_EXPERTS))


def _ffn_kernel(te_ref, nlive_ref, xs_ref, wg_ref, wu_ref, wd_ref, ys_ref):
    t = pl.program_id(0)

    @pl.when(t < nlive_ref[0])
    def _():
        x = xs_ref[...]
        wg = wg_ref[0]
        wu = wu_ref[0]
        wd = wd_ref[0]
        g = lax.dot_general(x, wg, (((1,), (1,)), ((), ())),
                            preferred_element_type=jnp.float32)
        u = lax.dot_general(x, wu, (((1,), (1,)), ((), ())),
                            preferred_element_type=jnp.float32)
        h = (g * jax.nn.sigmoid(g)) * u
        ys_ref[...] = lax.dot_general(h, wd, (((1,), (1,)), ((), ())),
                                      preferred_element_type=jnp.float32)


def _run_ffn(xs, w_gate, w_up, w_down, tile_expert, nlive, pp):
    nt = pp // TILE
    grid_spec = pltpu.PrefetchScalarGridSpec(
        num_scalar_prefetch=2,
        grid=(nt,),
        in_specs=[
            pl.BlockSpec((TILE, D_MODEL), lambda t, te, nl: (t, 0)),
            pl.BlockSpec((1, FFN, D_MODEL), lambda t, te, nl: (te[t], 0, 0)),
            pl.BlockSpec((1, FFN, D_MODEL), lambda t, te, nl: (te[t], 0, 0)),
            pl.BlockSpec((1, D_MODEL, FFN), lambda t, te, nl: (te[t], 0, 0)),
        ],
        out_specs=pl.BlockSpec((TILE, D_MODEL), lambda t, te, nl: (t, 0)),
    )
    return pl.pallas_call(
        _ffn_kernel,
        grid_spec=grid_spec,
        out_shape=jax.ShapeDtypeStruct((pp, D_MODEL), jnp.float32),
        compiler_params=pltpu.CompilerParams(
            vmem_limit_bytes=100 * 1024 * 1024),
    )(tile_expert, nlive, xs, w_gate, w_up, w_down)


def _dispatch_scatter_sc(xdup, slot_pair, pp):
    p2, d = xdup.shape
    per_w = p2 // NW
    ch = 64
    mesh = plsc.VectorSubcoreMesh(core_axis_name="c", subcore_axis_name="s")

    @functools.partial(
        pl.kernel, mesh=mesh,
        out_type=jax.ShapeDtypeStruct((pp, d), jnp.float32),
        scratch_types=[
            pltpu.VMEM((ch,), jnp.int32),
            pltpu.VMEM((ch, d), jnp.float32),
            pltpu.SemaphoreType.DMA,
        ],
    )
    def k(xdup_hbm, sp_hbm, xs_hbm, idx_v, rows_v, sem):
        wid = lax.axis_index("s") * SC_CORES + lax.axis_index("c")
        base = wid * per_w
        for j in range(per_w // ch):
            off = base + j * ch
            pltpu.sync_copy(sp_hbm.at[pl.ds(off, ch)], idx_v)
            pltpu.sync_copy(xdup_hbm.at[pl.ds(off, ch)], rows_v)
            pltpu.async_copy(rows_v, xs_hbm.at[idx_v], sem).wait()

    return k(xdup, slot_pair)


def _combine_sc(ys, s_a, s_b, wa16, wb16):
    n = s_a.shape[0]
    d = ys.shape[1]
    per_w = n // NW
    ch = 32
    mesh = plsc.VectorSubcoreMesh(core_axis_name="c", subcore_axis_name="s")

    @functools.partial(
        pl.kernel, mesh=mesh,
        out_type=jax.ShapeDtypeStruct((n, d), jnp.float32),
        scratch_types=[
            pltpu.VMEM((ch,), jnp.int32),
            pltpu.VMEM((ch,), jnp.int32),
            pltpu.VMEM((ch, d), jnp.float32),
            pltpu.VMEM((ch, d), jnp.float32),
            pltpu.VMEM((ch, 16), jnp.float32),
            pltpu.VMEM((ch, 16), jnp.float32),
            pltpu.SemaphoreType.DMA,
        ],
    )
    def k(ys_hbm, sa_hbm, sb_hbm, wa_hbm, wb_hbm, out_hbm,
          ia_v, ib_v, a_v, b_v, wa_v, wb_v, sem):
        wid = lax.axis_index("s") * SC_CORES + lax.axis_index("c")
        base = wid * per_w
        for j in range(per_w // ch):
            off = base + j * ch
            pltpu.sync_copy(sa_hbm.at[pl.ds(off, ch)], ia_v)
            pltpu.sync_copy(sb_hbm.at[pl.ds(off, ch)], ib_v)
            pltpu.sync_copy(wa_hbm.at[pl.ds(off, ch)], wa_v)
            pltpu.sync_copy(wb_hbm.at[pl.ds(off, ch)], wb_v)
            ca = pltpu.async_copy(ys_hbm.at[ia_v], a_v, sem)
            cb = pltpu.async_copy(ys_hbm.at[ib_v], b_v, sem)
            ca.wait()
            cb.wait()

            @pl.loop(0, ch)
            def _(r):
                wa = wa_v[r, :]
                wb = wb_v[r, :]

                @pl.loop(0, d, step=64)
                def _(c):
                    for u in range(4):
                        sl = pl.ds(c + u * 16, 16)
                        a_v[r, sl] = a_v[r, sl] * wa + b_v[r, sl] * wb

            pltpu.sync_copy(a_v, out_hbm.at[pl.ds(off, ch)])

    return k(ys, s_a, s_b, wa16, wb16)


def kernel(x, gate_w, w_gate, w_up, w_down, expert_bias):
    bb, ss, dd = x.shape
    n = bb * ss
    pp = TOP_K * n + N_EXPERTS * TILE
    flat_x = x.reshape(n, dd)

    meta, wa16, wb16, cntf = _run_router(flat_x, gate_w, expert_bias)

    counts = cntf[0].astype(jnp.int32)
    padded = ((counts + TILE - 1) // TILE) * TILE
    ends = jnp.cumsum(padded)
    poff = ends - padded
    poff_f = poff.astype(jnp.float32)
    eids = jnp.arange(N_EXPERTS, dtype=jnp.float32)
    off_a = jnp.sum((meta[:, 2:3] == eids[None, :]) * poff_f[None, :], axis=1)
    off_b = jnp.sum((meta[:, 3:4] == eids[None, :]) * poff_f[None, :], axis=1)
    s_a = (meta[:, 0] + off_a).astype(jnp.int32)
    s_b = (meta[:, 1] + off_b).astype(jnp.int32)
    slot_pair = jnp.stack([s_a, s_b], axis=1).reshape(-1)
    nt = pp // TILE
    tile_starts = jnp.arange(nt, dtype=jnp.int32) * TILE
    tile_expert = jnp.minimum(
        jnp.sum((tile_starts[:, None] >= ends[None, :]).astype(jnp.int32),
                axis=1), N_EXPERTS - 1).astype(jnp.int32)
    nlive = (ends[-1] // TILE).astype(jnp.int32).reshape(1)
    xdup = jnp.broadcast_to(flat_x[:, None, :], (n, TOP_K, dd)).reshape(
        TOP_K * n, dd)

    xs = _dispatch_scatter_sc(xdup, slot_pair, pp)

    ys = _run_ffn(xs, w_gate, w_up, w_down, tile_expert, nlive, pp)

    out = _combine_sc(ys, s_a, s_b, wa16, wb16)
    return out.reshape(bb, ss, dd)


# speedup vs baseline: 1.3433x; 1.3433x over previous
import functools

import jax
import jax.numpy as jnp
from jax import lax
from jax.experimental import pallas as pl
from jax.experimental.pallas import tpu as pltpu
from jax.experimental.pallas import tpu_sc as plsc

D_MODEL = 1024
FFN = 2048
N_EXPERTS = 8
TOP_K = 2
TILE = 256
ROW_TILE = 256

SC_CORES = 2
SC_SUBCORES = 16
NW = SC_CORES * SC_SUBCORES


def _router_kernel(x_ref, gw_ref, b_ref, meta_ref, wa_ref, wb_ref, cnt_ref,
                   run_ref):
    t = pl.program_id(0)

    @pl.when(t == 0)
    def _():
        run_ref[...] = jnp.zeros_like(run_ref)

    x = x_ref[...]
    gw = gw_ref[...]
    logits = lax.dot_general(x, gw, (((1,), (1,)), ((), ())),
                             preferred_element_type=jnp.float32)
    m = jnp.max(logits, axis=-1, keepdims=True)
    e = jnp.exp(logits - m)
    b = logits + b_ref[...]
    lane = lax.broadcasted_iota(jnp.int32, b.shape, 1)
    v1 = jnp.max(b, axis=-1, keepdims=True)
    i1 = jnp.min(jnp.where(b == v1, lane, N_EXPERTS), axis=-1, keepdims=True)
    oh1 = (lane == i1).astype(jnp.float32)
    b2 = jnp.where(oh1 > 0, -1e30, b)
    v2 = jnp.max(b2, axis=-1, keepdims=True)
    i2 = jnp.min(jnp.where(b2 == v2, lane, N_EXPERTS), axis=-1, keepdims=True)
    oh2 = (lane == i2).astype(jnp.float32)
    e1 = jnp.sum(e * oh1, axis=-1, keepdims=True)
    e2 = jnp.sum(e * oh2, axis=-1, keepdims=True)
    s = e1 + e2
    w1 = e1 / s
    w2 = e2 / s
    maskf = oh1 + oh2
    r_i = lax.broadcasted_iota(jnp.int32, (ROW_TILE, ROW_TILE), 0)
    c_i = lax.broadcasted_iota(jnp.int32, (ROW_TILE, ROW_TILE), 1)
    tri = (r_i >= c_i).astype(jnp.float32)
    local = lax.dot_general(tri, maskf, (((1,), (0,)), ((), ())),
                            preferred_element_type=jnp.float32)
    run0 = run_ref[...][0:1, :]
    rank = local + run0 - 1.0
    new_run = run0 + local[ROW_TILE - 1:ROW_TILE, :]
    run_ref[...] = jnp.broadcast_to(new_run, (N_EXPERTS, N_EXPERTS))
    cnt_ref[...] = jnp.broadcast_to(new_run, (N_EXPERTS, N_EXPERTS))
    i1f = i1.astype(jnp.float32)
    i2f = i2.astype(jnp.float32)
    i_lo = jnp.minimum(i1f, i2f)
    i_hi = jnp.maximum(i1f, i2f)
    lo_is_1 = (i1 < i2).astype(jnp.float32)
    w_lo = lo_is_1 * w1 + (1.0 - lo_is_1) * w2
    w_hi = lo_is_1 * w2 + (1.0 - lo_is_1) * w1
    oh_lo = (lane.astype(jnp.float32) == i_lo).astype(jnp.float32)
    oh_hi = (lane.astype(jnp.float32) == i_hi).astype(jnp.float32)
    rank_lo = jnp.sum(rank * oh_lo, axis=-1, keepdims=True)
    rank_hi = jnp.sum(rank * oh_hi, axis=-1, keepdims=True)
    l0 = (lane == 0).astype(jnp.float32)
    l1 = (lane == 1).astype(jnp.float32)
    l2 = (lane == 2).astype(jnp.float32)
    l3 = (lane == 3).astype(jnp.float32)
    meta_ref[...] = l0 * rank_lo + l1 * rank_hi + l2 * i_lo + l3 * i_hi
    wa_ref[...] = jnp.broadcast_to(w_lo, (ROW_TILE, 16))
    wb_ref[...] = jnp.broadcast_to(w_hi, (ROW_TILE, 16))


def _run_router(flat_x, gate_w, expert_bias):
    n = flat_x.shape[0]
    grid = (n // ROW_TILE,)
    return pl.pallas_call(
        _router_kernel,
        grid=grid,
        in_specs=[
            pl.BlockSpec((ROW_TILE, D_MODEL), lambda t: (t, 0)),
            pl.BlockSpec((N_EXPERTS, D_MODEL), lambda t: (0, 0)),
            pl.BlockSpec((1, N_EXPERTS), lambda t: (0, 0)),
        ],
        out_specs=[
            pl.BlockSpec((ROW_TILE, N_EXPERTS), lambda t: (t, 0)),
            pl.BlockSpec((ROW_TILE, 16), lambda t: (t, 0)),
            pl.BlockSpec((ROW_TILE, 16), lambda t: (t, 0)),
            pl.BlockSpec((N_EXPERTS, N_EXPERTS), lambda t: (0, 0)),
        ],
        out_shape=[
            jax.ShapeDtypeStruct((n, N_EXPERTS), jnp.float32),
            jax.ShapeDtypeStruct((n, 16), jnp.float32),
            jax.ShapeDtypeStruct((n, 16), jnp.float32),
            jax.ShapeDtypeStruct((N_EXPERTS, N_EXPERTS), jnp.float32),
        ],
        scratch_shapes=[pltpu.VMEM((N_EXPERTS, N_EXPERTS), jnp.float32)],
    )(flat_x, gate_w, expert_bias.reshape(1, N_EXPERTS))


def _ffn_kernel(te_ref, nlive_ref, xs_ref, wg_ref, wu_ref, wd_ref, ys_ref):
    t = pl.program_id(0)

    @pl.when(t < nlive_ref[0])
    def _():
        x = xs_ref[...]
        wg = wg_ref[0]
        wu = wu_ref[0]
        wd = wd_ref[0]
        g = lax.dot_general(x, wg, (((1,), (1,)), ((), ())),
                            preferred_element_type=jnp.float32)
        u = lax.dot_general(x, wu, (((1,), (1,)), ((), ())),
                            preferred_element_type=jnp.float32)
        h = (g * jax.nn.sigmoid(g)) * u
        ys_ref[...] = lax.dot_general(h, wd, (((1,), (1,)), ((), ())),
                                      preferred_element_type=jnp.float32)


def _run_ffn(xs, w_gate, w_up, w_down, tile_expert, nlive, pp):
    nt = pp // TILE
    grid_spec = pltpu.PrefetchScalarGridSpec(
        num_scalar_prefetch=2,
        grid=(nt,),
        in_specs=[
            pl.BlockSpec((TILE, D_MODEL), lambda t, te, nl: (t, 0)),
            pl.BlockSpec((1, FFN, D_MODEL), lambda t, te, nl: (te[t], 0, 0)),
            pl.BlockSpec((1, FFN, D_MODEL), lambda t, te, nl: (te[t], 0, 0)),
            pl.BlockSpec((1, D_MODEL, FFN), lambda t, te, nl: (te[t], 0, 0)),
        ],
        out_specs=pl.BlockSpec((TILE, D_MODEL), lambda t, te, nl: (t, 0)),
    )
    return pl.pallas_call(
        _ffn_kernel,
        grid_spec=grid_spec,
        out_shape=jax.ShapeDtypeStruct((pp, D_MODEL), jnp.float32),
        compiler_params=pltpu.CompilerParams(
            vmem_limit_bytes=100 * 1024 * 1024),
    )(tile_expert, nlive, xs, w_gate, w_up, w_down)


def _dispatch_scatter_sc(xdup, slot_pair, pp):
    p2, d = xdup.shape
    per_w = p2 // NW
    ch = 64
    mesh = plsc.VectorSubcoreMesh(core_axis_name="c", subcore_axis_name="s")

    @functools.partial(
        pl.kernel, mesh=mesh,
        out_type=jax.ShapeDtypeStruct((pp, d), jnp.float32),
        scratch_types=[
            pltpu.VMEM((ch,), jnp.int32),
            pltpu.VMEM((ch, d), jnp.float32),
            pltpu.SemaphoreType.DMA,
        ],
    )
    def k(xdup_hbm, sp_hbm, xs_hbm, idx_v, rows_v, sem):
        wid = lax.axis_index("s") * SC_CORES + lax.axis_index("c")
        base = wid * per_w
        for j in range(per_w // ch):
            off = base + j * ch
            pltpu.sync_copy(sp_hbm.at[pl.ds(off, ch)], idx_v)
            pltpu.sync_copy(xdup_hbm.at[pl.ds(off, ch)], rows_v)
            pltpu.async_copy(rows_v, xs_hbm.at[idx_v], sem).wait()

    return k(xdup, slot_pair)


def _combine_sc(ys, s_a, s_b, wa16, wb16):
    n = s_a.shape[0]
    d = ys.shape[1]
    per_w = n // NW
    ch = 32
    mesh = plsc.VectorSubcoreMesh(core_axis_name="c", subcore_axis_name="s")

    @functools.partial(
        pl.kernel, mesh=mesh,
        out_type=jax.ShapeDtypeStruct((n, d), jnp.float32),
        scratch_types=[
            pltpu.VMEM((ch,), jnp.int32),
            pltpu.VMEM((ch,), jnp.int32),
            pltpu.VMEM((ch, d), jnp.float32),
            pltpu.VMEM((ch, d), jnp.float32),
            pltpu.VMEM((ch, 16), jnp.float32),
            pltpu.VMEM((ch, 16), jnp.float32),
            pltpu.SemaphoreType.DMA,
        ],
    )
    def k(ys_hbm, sa_hbm, sb_hbm, wa_hbm, wb_hbm, out_hbm,
          ia_v, ib_v, a_v, b_v, wa_v, wb_v, sem):
        wid = lax.axis_index("s") * SC_CORES + lax.axis_index("c")
        base = wid * per_w
        for j in range(per_w // ch):
            off = base + j * ch
            pltpu.sync_copy(sa_hbm.at[pl.ds(off, ch)], ia_v)
            pltpu.sync_copy(sb_hbm.at[pl.ds(off, ch)], ib_v)
            pltpu.sync_copy(wa_hbm.at[pl.ds(off, ch)], wa_v)
            pltpu.sync_copy(wb_hbm.at[pl.ds(off, ch)], wb_v)
            ca = pltpu.async_copy(ys_hbm.at[ia_v], a_v, sem)
            cb = pltpu.async_copy(ys_hbm.at[ib_v], b_v, sem)
            ca.wait()
            cb.wait()

            @pl.loop(0, ch)
            def _(r):
                wa = wa_v[r, :]
                wb = wb_v[r, :]

                @pl.loop(0, d, step=64)
                def _(c):
                    for u in range(4):
                        sl = pl.ds(c + u * 16, 16)
                        a_v[r, sl] = a_v[r, sl] * wa + b_v[r, sl] * wb

            pltpu.sync_copy(a_v, out_hbm.at[pl.ds(off, ch)])

    return k(ys, s_a, s_b, wa16, wb16)


def kernel(x, gate_w, w_gate, w_up, w_down, expert_bias):
    bb, ss, dd = x.shape
    n = bb * ss
    pp = TOP_K * n + N_EXPERTS * TILE
    flat_x = x.reshape(n, dd)

    meta, wa16, wb16, cntf = _run_router(flat_x, gate_w, expert_bias)

    counts = cntf[0].astype(jnp.int32)
    padded = ((counts + TILE - 1) // TILE) * TILE
    ends = jnp.cumsum(padded)
    poff = ends - padded
    poff_f = poff.astype(jnp.float32)
    eids = jnp.arange(N_EXPERTS, dtype=jnp.float32)
    off_a = jnp.sum((meta[:, 2:3] == eids[None, :]) * poff_f[None, :], axis=1)
    off_b = jnp.sum((meta[:, 3:4] == eids[None, :]) * poff_f[None, :], axis=1)
    s_a = (meta[:, 0] + off_a).astype(jnp.int32)
    s_b = (meta[:, 1] + off_b).astype(jnp.int32)
    slot_pair = jnp.stack([s_a, s_b], axis=1).reshape(-1)
    nt = pp // TILE
    tile_starts = jnp.arange(nt, dtype=jnp.int32) * TILE
    tile_expert = jnp.minimum(
        jnp.sum((tile_starts[:, None] >= ends[None, :]).astype(jnp.int32),
                axis=1), N_EXPERTS - 1).astype(jnp.int32)
    nlive = (ends[-1] // TILE).astype(jnp.int32).reshape(1)
    xdup = jnp.broadcast_to(flat_x[:, None, :], (n, TOP_K, dd)).reshape(
        TOP_K * n, dd)

    xs = _dispatch_scatter_sc(xdup, slot_pair, pp)

    ys = _run_ffn(xs, w_gate, w_up, w_down, tile_expert, nlive, pp)

    out = _combine_sc(ys, s_a, s_b, wa16, wb16)
    return out.reshape(bb, ss, dd)


# speedup vs baseline: 1.4421x; 1.0735x over previous
import functools

import jax
import jax.numpy as jnp
from jax import lax
from jax.experimental import pallas as pl
from jax.experimental.pallas import tpu as pltpu
from jax.experimental.pallas import tpu_sc as plsc

D_MODEL = 1024
FFN = 2048
N_EXPERTS = 8
TOP_K = 2
TILE = 512
ROW_TILE = 256

SC_CORES = 2
SC_SUBCORES = 16
NW = SC_CORES * SC_SUBCORES


def _router_kernel(x_ref, gw_ref, b_ref, meta_ref, wa_ref, wb_ref, cnt_ref,
                   run_ref):
    t = pl.program_id(0)

    @pl.when(t == 0)
    def _():
        run_ref[...] = jnp.zeros_like(run_ref)

    x = x_ref[...]
    gw = gw_ref[...]
    logits = lax.dot_general(x, gw, (((1,), (1,)), ((), ())),
                             preferred_element_type=jnp.float32)
    m = jnp.max(logits, axis=-1, keepdims=True)
    e = jnp.exp(logits - m)
    b = logits + b_ref[...]
    lane = lax.broadcasted_iota(jnp.int32, b.shape, 1)
    v1 = jnp.max(b, axis=-1, keepdims=True)
    i1 = jnp.min(jnp.where(b == v1, lane, N_EXPERTS), axis=-1, keepdims=True)
    oh1 = (lane == i1).astype(jnp.float32)
    b2 = jnp.where(oh1 > 0, -1e30, b)
    v2 = jnp.max(b2, axis=-1, keepdims=True)
    i2 = jnp.min(jnp.where(b2 == v2, lane, N_EXPERTS), axis=-1, keepdims=True)
    oh2 = (lane == i2).astype(jnp.float32)
    e1 = jnp.sum(e * oh1, axis=-1, keepdims=True)
    e2 = jnp.sum(e * oh2, axis=-1, keepdims=True)
    s = e1 + e2
    w1 = e1 / s
    w2 = e2 / s
    maskf = oh1 + oh2
    r_i = lax.broadcasted_iota(jnp.int32, (ROW_TILE, ROW_TILE), 0)
    c_i = lax.broadcasted_iota(jnp.int32, (ROW_TILE, ROW_TILE), 1)
    tri = (r_i >= c_i).astype(jnp.float32)
    local = lax.dot_general(tri, maskf, (((1,), (0,)), ((), ())),
                            preferred_element_type=jnp.float32)
    run0 = run_ref[...][0:1, :]
    rank = local + run0 - 1.0
    new_run = run0 + local[ROW_TILE - 1:ROW_TILE, :]
    run_ref[...] = jnp.broadcast_to(new_run, (N_EXPERTS, N_EXPERTS))
    cnt_ref[...] = jnp.broadcast_to(new_run, (N_EXPERTS, N_EXPERTS))
    i1f = i1.astype(jnp.float32)
    i2f = i2.astype(jnp.float32)
    i_lo = jnp.minimum(i1f, i2f)
    i_hi = jnp.maximum(i1f, i2f)
    lo_is_1 = (i1 < i2).astype(jnp.float32)
    w_lo = lo_is_1 * w1 + (1.0 - lo_is_1) * w2
    w_hi = lo_is_1 * w2 + (1.0 - lo_is_1) * w1
    oh_lo = (lane.astype(jnp.float32) == i_lo).astype(jnp.float32)
    oh_hi = (lane.astype(jnp.float32) == i_hi).astype(jnp.float32)
    rank_lo = jnp.sum(rank * oh_lo, axis=-1, keepdims=True)
    rank_hi = jnp.sum(rank * oh_hi, axis=-1, keepdims=True)
    l0 = (lane == 0).astype(jnp.float32)
    l1 = (lane == 1).astype(jnp.float32)
    l2 = (lane == 2).astype(jnp.float32)
    l3 = (lane == 3).astype(jnp.float32)
    meta_ref[...] = l0 * rank_lo + l1 * rank_hi + l2 * i_lo + l3 * i_hi
    wa_ref[...] = jnp.broadcast_to(w_lo, (ROW_TILE, 16))
    wb_ref[...] = jnp.broadcast_to(w_hi, (ROW_TILE, 16))


def _run_router(flat_x, gate_w, expert_bias):
    n = flat_x.shape[0]
    grid = (n // ROW_TILE,)
    return pl.pallas_call(
        _router_kernel,
        grid=grid,
        in_specs=[
            pl.BlockSpec((ROW_TILE, D_MODEL), lambda t: (t, 0)),
            pl.BlockSpec((N_EXPERTS, D_MODEL), lambda t: (0, 0)),
            pl.BlockSpec((1, N_EXPERTS), lambda t: (0, 0)),
        ],
        out_specs=[
            pl.BlockSpec((ROW_TILE, N_EXPERTS), lambda t: (t, 0)),
            pl.BlockSpec((ROW_TILE, 16), lambda t: (t, 0)),
            pl.BlockSpec((ROW_TILE, 16), lambda t: (t, 0)),
            pl.BlockSpec((N_EXPERTS, N_EXPERTS), lambda t: (0, 0)),
        ],
        out_shape=[
            jax.ShapeDtypeStruct((n, N_EXPERTS), jnp.float32),
            jax.ShapeDtypeStruct((n, 16), jnp.float32),
            jax.ShapeDtypeStruct((n, 16), jnp.float32),
            jax.ShapeDtypeStruct((N_EXPERTS, N_EXPERTS), jnp.float32),
        ],
        scratch_shapes=[pltpu.VMEM((N_EXPERTS, N_EXPERTS), jnp.float32)],
    )(flat_x, gate_w, expert_bias.reshape(1, N_EXPERTS))


def _ffn_kernel(te_ref, nlive_ref, xs_ref, wg_ref, wu_ref, wd_ref, ys_ref):
    t = pl.program_id(0)

    @pl.when(t < nlive_ref[0])
    def _():
        x = xs_ref[...]
        wg = wg_ref[0]
        wu = wu_ref[0]
        wd = wd_ref[0]
        g = lax.dot_general(x, wg, (((1,), (1,)), ((), ())),
                            preferred_element_type=jnp.float32)
        u = lax.dot_general(x, wu, (((1,), (1,)), ((), ())),
                            preferred_element_type=jnp.float32)
        h = (g * jax.nn.sigmoid(g)) * u
        ys_ref[...] = lax.dot_general(h, wd, (((1,), (1,)), ((), ())),
                                      preferred_element_type=jnp.float32)


def _run_ffn(xs, w_gate, w_up, w_down, tile_expert, nlive, pp):
    nt = pp // TILE
    grid_spec = pltpu.PrefetchScalarGridSpec(
        num_scalar_prefetch=2,
        grid=(nt,),
        in_specs=[
            pl.BlockSpec((TILE, D_MODEL), lambda t, te, nl: (t, 0)),
            pl.BlockSpec((1, FFN, D_MODEL), lambda t, te, nl: (te[t], 0, 0)),
            pl.BlockSpec((1, FFN, D_MODEL), lambda t, te, nl: (te[t], 0, 0)),
            pl.BlockSpec((1, D_MODEL, FFN), lambda t, te, nl: (te[t], 0, 0)),
        ],
        out_specs=pl.BlockSpec((TILE, D_MODEL), lambda t, te, nl: (t, 0)),
    )
    return pl.pallas_call(
        _ffn_kernel,
        grid_spec=grid_spec,
        out_shape=jax.ShapeDtypeStruct((pp, D_MODEL), jnp.float32),
        compiler_params=pltpu.CompilerParams(
            vmem_limit_bytes=100 * 1024 * 1024),
    )(tile_expert, nlive, xs, w_gate, w_up, w_down)


def _dispatch_scatter_sc(xdup, slot_pair, pp):
    p2, d = xdup.shape
    per_w = p2 // NW
    ch = 64
    mesh = plsc.VectorSubcoreMesh(core_axis_name="c", subcore_axis_name="s")

    @functools.partial(
        pl.kernel, mesh=mesh,
        out_type=jax.ShapeDtypeStruct((pp, d), jnp.float32),
        scratch_types=[
            pltpu.VMEM((ch,), jnp.int32),
            pltpu.VMEM((ch, d), jnp.float32),
            pltpu.SemaphoreType.DMA,
        ],
    )
    def k(xdup_hbm, sp_hbm, xs_hbm, idx_v, rows_v, sem):
        wid = lax.axis_index("s") * SC_CORES + lax.axis_index("c")
        base = wid * per_w
        for j in range(per_w // ch):
            off = base + j * ch
            pltpu.sync_copy(sp_hbm.at[pl.ds(off, ch)], idx_v)
            pltpu.sync_copy(xdup_hbm.at[pl.ds(off, ch)], rows_v)
            pltpu.async_copy(rows_v, xs_hbm.at[idx_v], sem).wait()

    return k(xdup, slot_pair)


def _combine_sc(ys, s_a, s_b, wa16, wb16):
    n = s_a.shape[0]
    d = ys.shape[1]
    per_w = n // NW
    ch = 32
    mesh = plsc.VectorSubcoreMesh(core_axis_name="c", subcore_axis_name="s")

    @functools.partial(
        pl.kernel, mesh=mesh,
        out_type=jax.ShapeDtypeStruct((n, d), jnp.float32),
        scratch_types=[
            pltpu.VMEM((ch,), jnp.int32),
            pltpu.VMEM((ch,), jnp.int32),
            pltpu.VMEM((ch, d), jnp.float32),
            pltpu.VMEM((ch, d), jnp.float32),
            pltpu.VMEM((ch, 16), jnp.float32),
            pltpu.VMEM((ch, 16), jnp.float32),
            pltpu.SemaphoreType.DMA,
        ],
    )
    def k(ys_hbm, sa_hbm, sb_hbm, wa_hbm, wb_hbm, out_hbm,
          ia_v, ib_v, a_v, b_v, wa_v, wb_v, sem):
        wid = lax.axis_index("s") * SC_CORES + lax.axis_index("c")
        base = wid * per_w
        for j in range(per_w // ch):
            off = base + j * ch
            pltpu.sync_copy(sa_hbm.at[pl.ds(off, ch)], ia_v)
            pltpu.sync_copy(sb_hbm.at[pl.ds(off, ch)], ib_v)
            pltpu.sync_copy(wa_hbm.at[pl.ds(off, ch)], wa_v)
            pltpu.sync_copy(wb_hbm.at[pl.ds(off, ch)], wb_v)
            ca = pltpu.async_copy(ys_hbm.at[ia_v], a_v, sem)
            cb = pltpu.async_copy(ys_hbm.at[ib_v], b_v, sem)
            ca.wait()
            cb.wait()

            @pl.loop(0, ch)
            def _(r):
                wa = wa_v[r, :]
                wb = wb_v[r, :]

                @pl.loop(0, d, step=64)
                def _(c):
                    for u in range(4):
                        sl = pl.ds(c + u * 16, 16)
                        a_v[r, sl] = a_v[r, sl] * wa + b_v[r, sl] * wb

            pltpu.sync_copy(a_v, out_hbm.at[pl.ds(off, ch)])

    return k(ys, s_a, s_b, wa16, wb16)


def kernel(x, gate_w, w_gate, w_up, w_down, expert_bias):
    bb, ss, dd = x.shape
    n = bb * ss
    pp = TOP_K * n + N_EXPERTS * TILE
    flat_x = x.reshape(n, dd)

    meta, wa16, wb16, cntf = _run_router(flat_x, gate_w, expert_bias)

    counts = cntf[0].astype(jnp.int32)
    padded = ((counts + TILE - 1) // TILE) * TILE
    ends = jnp.cumsum(padded)
    poff = ends - padded
    poff_f = poff.astype(jnp.float32)
    eids = jnp.arange(N_EXPERTS, dtype=jnp.float32)
    off_a = jnp.sum((meta[:, 2:3] == eids[None, :]) * poff_f[None, :], axis=1)
    off_b = jnp.sum((meta[:, 3:4] == eids[None, :]) * poff_f[None, :], axis=1)
    s_a = (meta[:, 0] + off_a).astype(jnp.int32)
    s_b = (meta[:, 1] + off_b).astype(jnp.int32)
    slot_pair = jnp.stack([s_a, s_b], axis=1).reshape(-1)
    nt = pp // TILE
    tile_starts = jnp.arange(nt, dtype=jnp.int32) * TILE
    tile_expert = jnp.minimum(
        jnp.sum((tile_starts[:, None] >= ends[None, :]).astype(jnp.int32),
                axis=1), N_EXPERTS - 1).astype(jnp.int32)
    nlive = (ends[-1] // TILE).astype(jnp.int32).reshape(1)
    xdup = jnp.broadcast_to(flat_x[:, None, :], (n, TOP_K, dd)).reshape(
        TOP_K * n, dd)

    xs = _dispatch_scatter_sc(xdup, slot_pair, pp)

    ys = _run_ffn(xs, w_gate, w_up, w_down, tile_expert, nlive, pp)

    out = _combine_sc(ys, s_a, s_b, wa16, wb16)
    return out.reshape(bb, ss, dd)
